# baseline (speedup 1.0000x reference)
import functools

import jax
import jax.numpy as jnp
from jax import lax
from jax.experimental import pallas as pl
from jax.experimental.pallas import tpu as pltpu
from jax.experimental.pallas import tpu_sc as plsc

_NC = 2
_NS = 16
_LANES = 16


def _sc_aggregate(prod2, src3, dst3, ew3):
    n2, dh = prod2.shape
    n = n2 // 2
    _, n_chunks, chunk = src3.shape
    e_per_tile = n_chunks * chunk
    assert n_chunks % 2 == 0
    n_pad = -(-n // (8 * _NS)) * (8 * _NS)
    rows_per_tile = n_pad // _NS
    n_vregs = dh // _LANES
    zr = rows_per_tile // 4

    mesh = plsc.VectorSubcoreMesh(core_axis_name="c", subcore_axis_name="s")

    @functools.partial(
        pl.kernel,
        out_type=jax.ShapeDtypeStruct((_NC, n_pad, dh), jnp.float32),
        mesh=mesh,
        compiler_params=pltpu.CompilerParams(
            use_tc_tiling_on_sc=False, needs_layout_passes=False),
        scratch_types=[
            pltpu.VMEM_SHARED((n_pad, dh), jnp.float32),
            pltpu.VMEM((n_chunks, chunk), jnp.int32),
            pltpu.VMEM((n_chunks, chunk), jnp.int32),
            pltpu.VMEM((n_chunks, chunk), jnp.float32),
            pltpu.VMEM((2, chunk, dh), jnp.bfloat16),
            pltpu.VMEM((2, chunk, dh), jnp.float32),
            pltpu.VMEM((zr, dh), jnp.float32),
            pltpu.SemaphoreType.DMA,
            pltpu.SemaphoreType.DMA,
            pltpu.SemaphoreType.DMA,
            pltpu.SemaphoreType.DMA,
        ],
    )
    def agg(src_hbm, dst_hbm, ew_hbm, prod_hbm, out_hbm,
            acc, idx_s, idx_d, w_v, gbuf, sbuf, zbuf,
            sg0, sg1, ss0, ss1):
        c = lax.axis_index("c")
        s = lax.axis_index("s")
        sg = (sg0, sg1)
        ss = (ss0, ss1)

        d_src = pltpu.async_copy(src_hbm.at[s], idx_s, sg0)
        d_dst = pltpu.async_copy(dst_hbm.at[s], idx_d, sg1)
        d_ew = pltpu.async_copy(ew_hbm.at[s], w_v, ss0)

        zeros16 = jnp.zeros((_LANES,), jnp.float32)

        @pl.loop(0, zr)
        def _zero_fill(r):
            for j in range(n_vregs):
                zbuf[r, pl.ds(j * _LANES, _LANES)] = zeros16

        stripe = s * rows_per_tile
        for q in range(4):
            pltpu.sync_copy(zbuf, acc.at[pl.ds(stripe + q * zr, zr), :])

        d_src.wait()

        @pl.loop(0, n_chunks)
        def _off(r):
            for k in range(chunk // _LANES):
                sl = pl.ds(k * _LANES, _LANES)
                idx_s[r, sl] = idx_s[r, sl] * 2 + c

        d_dst.wait()
        d_ew.wait()
        plsc.subcore_barrier()

        def start_gather(g, b):
            pltpu.async_copy(prod_hbm.at[idx_s.at[g]], gbuf.at[b], sg[b])

        def wait_gather(g, b):
            pltpu.make_async_copy(
                prod_hbm.at[idx_s.at[g]], gbuf.at[b], sg[b]).wait()

        def start_scatter(g, b):
            pltpu.async_copy(sbuf.at[b], acc.at[idx_d.at[g]], ss[b],
                             add=True)

        def wait_scatter(g, b):
            pltpu.make_async_copy(
                sbuf.at[b], acc.at[idx_d.at[g]], ss[b]).wait()

        start_gather(0, 0)
        start_gather(1, 1)

        dnums = lax.GatherDimensionNumbers(
            offset_dims=(), collapsed_slice_dims=(0,),
            start_index_map=(0,))

        @pl.loop(0, n_chunks, step=2)
        def _chunk(g0):
            for b in range(2):
                g = g0 + b
                wait_gather(g, b)

                @pl.when(g >= 2)
                def _():
                    wait_scatter(g - 2, b)

                for k in range(chunk // _LANES):
                    w16 = w_v[g, pl.ds(k * _LANES, _LANES)]
                    eb = k * _LANES
                    for i in range(_LANES):
                        lane = jnp.full((_LANES, 1), i, jnp.int32)
                        wv = lax.gather(
                            w16, lane, dnums, slice_sizes=(1,),
                            mode=lax.GatherScatterMode.PROMISE_IN_BOUNDS)
                        for h in range(n_vregs // 2):
                            v = gbuf[b, eb + i, pl.ds(h * 2 * _LANES,
                                                      2 * _LANES)]
                            lo, hi = plsc.unpack(
                                v, format=plsc.PackFormat.INTERLEAVED)
                            base = h * 2 * _LANES
                            sbuf[b, eb + i, pl.ds(base, _LANES)] = lo * wv
                            sbuf[b, eb + i, pl.ds(base + _LANES, _LANES)] = (
                                hi * wv)

                @pl.when(g + 2 < n_chunks)
                def _():
                    start_gather(g + 2, b)

                start_scatter(g, b)

        for b in range(2):
            wait_scatter(n_chunks - 2 + b, b)

        plsc.subcore_barrier()
        pltpu.sync_copy(acc.at[pl.ds(stripe, rows_per_tile), :],
                        out_hbm.at[c, pl.ds(stripe, rows_per_tile), :])

    return agg(src3, dst3, ew3, prod2)


def _epilogue(partials, n_norm, w):
    dh = partials.shape[2]
    n = n_norm.shape[0]
    d = 2 * dh
    bm = 2000

    def body(p_ref, nn_ref, w_ref, o_ref):
        h = jnp.concatenate([p_ref[0], p_ref[1]], axis=1)
        prod = jnp.dot(h, w_ref[...], preferred_element_type=jnp.float32)
        o_ref[...] = jnp.maximum(prod * nn_ref[...], 0.0)

    return pl.pallas_call(
        body,
        grid=(n // bm,),
        in_specs=[
            pl.BlockSpec((2, bm, dh), lambda i: (0, i, 0)),
            pl.BlockSpec((bm, 1), lambda i: (i, 0)),
            pl.BlockSpec((d, d), lambda i: (0, 0)),
        ],
        out_specs=pl.BlockSpec((bm, d), lambda i: (i, 0)),
        out_shape=jax.ShapeDtypeStruct((n, d), jnp.float32),
    )(partials, n_norm, w)


def kernel(x, edge_index, edge_weight, n_norm, W):
    e_total = edge_index.shape[1]
    chunk = 80
    e_per_tile = e_total // _NS
    n_chunks = e_per_tile // chunk
    dh = x.shape[1] // 2
    x2 = x.astype(jnp.bfloat16).reshape(-1, dh)
    src3 = edge_index[0].reshape(_NS, n_chunks, chunk)
    dst3 = edge_index[1].reshape(_NS, n_chunks, chunk)
    ew3 = edge_weight.reshape(_NS, n_chunks, chunk)
    partials = _sc_aggregate(x2, src3, dst3, ew3)
    perm = []
    for c in range(2):
        for h in range(dh // 32):
            for q in range(2):
                for p in range(_LANES):
                    perm.append(64 * c + 32 * h + 2 * p + q)
    Wp = W[jnp.array(perm, dtype=jnp.int32), :]
    return _epilogue(partials, n_norm, Wp)


# speedup vs baseline: 1.0044x; 1.0044x over previous
import functools

import jax
import jax.numpy as jnp
from jax import lax
from jax.experimental import pallas as pl
from jax.experimental.pallas import tpu as pltpu
from jax.experimental.pallas import tpu_sc as plsc

_NC = 2
_NS = 16
_LANES = 16


def _sc_aggregate(prod2, src3, dst3, ew3):
    n2, dh = prod2.shape
    n = n2 // 2
    _, n_chunks, chunk = src3.shape
    e_per_tile = n_chunks * chunk
    assert n_chunks % 2 == 0
    n_pad = -(-n // (8 * _NS)) * (8 * _NS)
    rows_per_tile = n_pad // _NS
    n_vregs = dh // _LANES
    zr = rows_per_tile // 4

    mesh = plsc.VectorSubcoreMesh(core_axis_name="c", subcore_axis_name="s")

    @functools.partial(
        pl.kernel,
        out_type=jax.ShapeDtypeStruct((_NC, n_pad, dh), jnp.float32),
        mesh=mesh,
        compiler_params=pltpu.CompilerParams(
            use_tc_tiling_on_sc=False, needs_layout_passes=False),
        scratch_types=[
            pltpu.VMEM_SHARED((n_pad, dh), jnp.float32),
            pltpu.VMEM((n_chunks, chunk), jnp.int32),
            pltpu.VMEM((n_chunks, chunk), jnp.int32),
            pltpu.VMEM((n_chunks, chunk), jnp.float32),
            pltpu.VMEM((2, chunk, dh), jnp.bfloat16),
            pltpu.VMEM((2, chunk, dh), jnp.float32),
            pltpu.VMEM((zr, dh), jnp.float32),
            pltpu.SemaphoreType.DMA,
            pltpu.SemaphoreType.DMA,
            pltpu.SemaphoreType.DMA,
            pltpu.SemaphoreType.DMA,
        ],
    )
    def agg(src_hbm, dst_hbm, ew_hbm, prod_hbm, out_hbm,
            acc, idx_s, idx_d, w_v, gbuf, sbuf, zbuf,
            sg0, sg1, ss0, ss1):
        c = lax.axis_index("c")
        s = lax.axis_index("s")
        sg = (sg0, sg1)
        ss = (ss0, ss1)

        d_src = pltpu.async_copy(src_hbm.at[s], idx_s, sg0)
        d_dst = pltpu.async_copy(dst_hbm.at[s], idx_d, sg1)
        d_ew = pltpu.async_copy(ew_hbm.at[s], w_v, ss0)

        zeros16 = jnp.zeros((_LANES,), jnp.float32)

        @pl.loop(0, zr)
        def _zero_fill(r):
            for j in range(n_vregs):
                zbuf[r, pl.ds(j * _LANES, _LANES)] = zeros16

        stripe = s * rows_per_tile
        for q in range(4):
            pltpu.sync_copy(zbuf, acc.at[pl.ds(stripe + q * zr, zr), :])

        d_src.wait()

        def _off_row(r):
            for k in range(chunk // _LANES):
                sl = pl.ds(k * _LANES, _LANES)
                idx_s[r, sl] = idx_s[r, sl] * 2 + c

        def start_gather(g, b):
            pltpu.async_copy(prod_hbm.at[idx_s.at[g]], gbuf.at[b], sg[b])

        def wait_gather(g, b):
            pltpu.make_async_copy(
                prod_hbm.at[idx_s.at[g]], gbuf.at[b], sg[b]).wait()

        def start_scatter(g, b):
            pltpu.async_copy(sbuf.at[b], acc.at[idx_d.at[g]], ss[b],
                             add=True)

        def wait_scatter(g, b):
            pltpu.make_async_copy(
                sbuf.at[b], acc.at[idx_d.at[g]], ss[b]).wait()

        for b in range(2):
            _off_row(b)
            start_gather(b, b)

        @pl.loop(2, n_chunks)
        def _off(r):
            _off_row(r)

        d_dst.wait()
        d_ew.wait()
        plsc.subcore_barrier()

        dnums = lax.GatherDimensionNumbers(
            offset_dims=(), collapsed_slice_dims=(0,),
            start_index_map=(0,))

        @pl.loop(0, n_chunks, step=2)
        def _chunk(g0):
            for b in range(2):
                g = g0 + b
                wait_gather(g, b)

                @pl.when(g >= 2)
                def _():
                    wait_scatter(g - 2, b)

                for k in range(chunk // _LANES):
                    w16 = w_v[g, pl.ds(k * _LANES, _LANES)]
                    eb = k * _LANES
                    for i in range(_LANES):
                        lane = jnp.full((_LANES, 1), i, jnp.int32)
                        wv = lax.gather(
                            w16, lane, dnums, slice_sizes=(1,),
                            mode=lax.GatherScatterMode.PROMISE_IN_BOUNDS)
                        for h in range(n_vregs // 2):
                            v = gbuf[b, eb + i, pl.ds(h * 2 * _LANES,
                                                      2 * _LANES)]
                            lo, hi = plsc.unpack(
                                v, format=plsc.PackFormat.INTERLEAVED)
                            base = h * 2 * _LANES
                            sbuf[b, eb + i, pl.ds(base, _LANES)] = lo * wv
                            sbuf[b, eb + i, pl.ds(base + _LANES, _LANES)] = (
                                hi * wv)

                @pl.when(g + 2 < n_chunks)
                def _():
                    start_gather(g + 2, b)

                start_scatter(g, b)

        for b in range(2):
            wait_scatter(n_chunks - 2 + b, b)

        plsc.subcore_barrier()
        pltpu.sync_copy(acc.at[pl.ds(stripe, rows_per_tile), :],
                        out_hbm.at[c, pl.ds(stripe, rows_per_tile), :])

    return agg(src3, dst3, ew3, prod2)


def _epilogue(partials, n_norm, w):
    dh = partials.shape[2]
    n = n_norm.shape[0]
    d = 2 * dh
    bm = 2000

    def body(p_ref, nn_ref, w_ref, o_ref):
        h = jnp.concatenate([p_ref[0], p_ref[1]], axis=1)
        prod = jnp.dot(h, w_ref[...], preferred_element_type=jnp.float32)
        o_ref[...] = jnp.maximum(prod * nn_ref[...], 0.0)

    return pl.pallas_call(
        body,
        grid=(n // bm,),
        in_specs=[
            pl.BlockSpec((2, bm, dh), lambda i: (0, i, 0)),
            pl.BlockSpec((bm, 1), lambda i: (i, 0)),
            pl.BlockSpec((d, d), lambda i: (0, 0)),
        ],
        out_specs=pl.BlockSpec((bm, d), lambda i: (i, 0)),
        out_shape=jax.ShapeDtypeStruct((n, d), jnp.float32),
    )(partials, n_norm, w)


def kernel(x, edge_index, edge_weight, n_norm, W):
    e_total = edge_index.shape[1]
    chunk = 80
    e_per_tile = e_total // _NS
    n_chunks = e_per_tile // chunk
    dh = x.shape[1] // 2
    x2 = x.astype(jnp.bfloat16).reshape(-1, dh)
    src3 = edge_index[0].reshape(_NS, n_chunks, chunk)
    dst3 = edge_index[1].reshape(_NS, n_chunks, chunk)
    ew3 = edge_weight.reshape(_NS, n_chunks, chunk)
    partials = _sc_aggregate(x2, src3, dst3, ew3)
    perm = []
    for c in range(2):
        for h in range(dh // 32):
            for q in range(2):
                for p in range(_LANES):
                    perm.append(64 * c + 32 * h + 2 * p + q)
    Wp = W[jnp.array(perm, dtype=jnp.int32), :]
    return _epilogue(partials, n_norm, Wp)
